# baseline (speedup 1.0000x reference)
import functools

import jax
import jax.numpy as jnp
from jax import lax
from jax.experimental import pallas as pl
from jax.experimental.pallas import tpu as pltpu
from jax.experimental.pallas import tpu_sc as plsc

L = 16


def _transpose_block(blk_v, line_v, iota, ncols):
    for c in range(ncols):
        col = plsc.load_gather(blk_v, [iota, jnp.full((L,), c, jnp.int32)])
        li, m = c // 8, c % 8
        plsc.store_scatter(
            line_v, [jnp.full((L,), li, jnp.int32), m * L + iota], col)


@functools.lru_cache(maxsize=None)
def _build_transpose(V, K):
    info = plsc.get_sparse_core_info()
    NC, NS = info.num_cores, info.num_subcores
    NW = NC * NS
    Vpad = -(-V // 128) * 128
    n_lines = Vpad // 8
    n_tc = Vpad // 128
    full_tc = n_tc - 1
    T = 2
    n_groups = full_tc // T
    n_iters = -(-n_groups // NW)
    n_pairs = -(-n_iters // 2)
    mesh = plsc.VectorSubcoreMesh(core_axis_name="c", subcore_axis_name="s")

    @functools.partial(
        pl.kernel,
        mesh=mesh,
        out_type=jax.ShapeDtypeStruct((n_lines, 128), jnp.float32),
        compiler_params=pltpu.CompilerParams(
            use_tc_tiling_on_sc=True, needs_layout_passes=False,
            disable_bounds_checks=True),
        scratch_types=[
            pltpu.VMEM((K, T * 128), jnp.float32),
            pltpu.VMEM((K, T * 128), jnp.float32),
            pltpu.VMEM((T * 16, 128), jnp.float32),
            pltpu.VMEM((T * 16, 128), jnp.float32),
            pltpu.SemaphoreType.DMA,
            pltpu.SemaphoreType.DMA,
            pltpu.SemaphoreType.DMA,
            pltpu.SemaphoreType.DMA,
        ],
    )
    def transpose_kernel(tab_hbm, tail_hbm, r_hbm,
                         blk_a, blk_b, line_a, line_b,
                         sin_a, sin_b, sout_a, sout_b):
        wid = lax.axis_index("s") * NC + lax.axis_index("c")
        iota = lax.iota(jnp.int32, L)
        blks = (blk_a, blk_b)
        lines = (line_a, line_b)
        sins = (sin_a, sin_b)
        souts = (sout_a, sout_b)

        def fire_in(j, buf, sem):
            g = j * NW + wid

            @pl.when(g < n_groups)
            def _():
                pltpu.async_copy(
                    tab_hbm.at[:, pl.ds(g * (T * 128), T * 128)], buf, sem)

        fire_in(0, blk_a, sin_a)

        def body(p, carry):
            for par in range(2):
                j = p * 2 + par
                g = j * NW + wid
                fire_in(j + 1, blks[1 - par], sins[1 - par])

                @pl.when(g < n_groups)
                def _(par=par, g=g):
                    pltpu.make_async_copy(
                        tab_hbm.at[:, pl.ds(g * (T * 128), T * 128)],
                        blks[par], sins[par]).wait()

                    @pl.when(p > 0)
                    def _():
                        pltpu.make_async_copy(
                            lines[par],
                            r_hbm.at[pl.ds(0, T * 16), :],
                            souts[par]).wait()

                    _transpose_block(blks[par], lines[par], iota, T * 128)
                    pltpu.async_copy(
                        lines[par],
                        r_hbm.at[pl.ds(g * T * 16, T * 16), :], souts[par])

            return carry

        lax.fori_loop(0, n_pairs, body, 0)
        for par in range(2):
            j_last = (n_pairs - 1) * 2 + par
            g_last = j_last * NW + wid

            @pl.when(g_last < n_groups)
            def _(par=par, g_last=g_last):
                pltpu.make_async_copy(
                    lines[par],
                    r_hbm.at[pl.ds(g_last * T * 16, T * 16), :],
                    souts[par]).wait()

        @pl.when(wid == 1)
        def _():
            pltpu.sync_copy(tail_hbm,
                            r_hbm.at[pl.ds(full_tc * 16, 16), :])

    return transpose_kernel


@functools.lru_cache(maxsize=None)
def _build_gather(B, F, V, K):
    info = plsc.get_sparse_core_info()
    NC, NS = info.num_cores, info.num_subcores
    NW = NC * NS
    assert B % (NW * 128) == 0 and K == L
    CB = B // NW
    TC = CB // 128
    G = 128
    n_sub = CB // G

    mesh = plsc.VectorSubcoreMesh(core_axis_name="c", subcore_axis_name="s")

    @functools.partial(
        pl.kernel,
        mesh=mesh,
        out_type=jax.ShapeDtypeStruct((F, K // 8, B // 128, 8, 128),
                                      jnp.float32),
        compiler_params=pltpu.CompilerParams(
            use_tc_tiling_on_sc=False, needs_layout_passes=False),
        scratch_types=[
            pltpu.VMEM((CB,), jnp.int32),
            pltpu.VMEM((CB,), jnp.float32),
            pltpu.VMEM((CB, K), jnp.float32),
            pltpu.VMEM((K // 8, TC, 8, 128), jnp.float32),
            pltpu.SemaphoreType.DMA,
        ],
    )
    def sc_kernel(idx_hbm, val_hbm, table_hbm, out_hbm,
                  idx_v, val_v, rows_v, outp_v, sem):
        wid = lax.axis_index("s") * NC + lax.axis_index("c")
        b0 = wid * CB
        iota = lax.iota(jnp.int32, L)

        def fbody(f, carry):
            pltpu.sync_copy(idx_hbm.at[f, pl.ds(b0, CB)], idx_v)
            pltpu.sync_copy(val_hbm.at[f, pl.ds(b0, CB)], val_v)
            cps = [
                pltpu.async_copy(
                    table_hbm.at[idx_v.at[pl.ds(g * G, G)]],
                    rows_v.at[pl.ds(g * G, G)],
                    sem,
                )
                for g in range(n_sub)
            ]
            for cp in cps:
                cp.wait()
            for j in range(CB // L):
                r16 = j * L + iota
                val16 = val_v[pl.ds(j * L, L)]
                for k in range(K):
                    col = plsc.load_gather(
                        rows_v, [r16, jnp.full((L,), k, jnp.int32)])
                    outp_v[k // 8, j // 8, k % 8,
                           pl.ds((j % 8) * L, L)] = col * val16
            pltpu.sync_copy(
                outp_v,
                out_hbm.at[f, :, pl.ds(wid * TC, TC), :, :])
            return carry

        lax.fori_loop(0, F, fbody, 0)

    return sc_kernel


def kernel(nonzero_index, nonzero_value, table):
    B, F = nonzero_index.shape
    V, K = table.shape
    Vpad = -(-V // 128) * 128
    table_rows = jnp.pad(table, ((0, Vpad - V), (0, 0)))
    idxt = nonzero_index.T.astype(jnp.int32)
    valt = nonzero_value.T
    res = _build_gather(B, F, Vpad, K)(idxt, valt, table_rows)
    out = res.transpose((2, 4, 0, 1, 3)).reshape(B, F, K)
    return out


# speedup vs baseline: 1.4238x; 1.4238x over previous
import functools

import jax
import jax.numpy as jnp
from jax import lax
from jax.experimental import pallas as pl
from jax.experimental.pallas import tpu as pltpu
from jax.experimental.pallas import tpu_sc as plsc

L = 16


@functools.lru_cache(maxsize=None)
def _build_gather(B, F, V, K):
    info = plsc.get_sparse_core_info()
    NC, NS = info.num_cores, info.num_subcores
    NW = NC * NS
    assert B % (NW * 128) == 0 and K == L
    CB = B // NW
    TC = CB // 128
    G = 128
    n_sub = CB // G

    mesh = plsc.VectorSubcoreMesh(core_axis_name="c", subcore_axis_name="s")

    @functools.partial(
        pl.kernel,
        mesh=mesh,
        out_type=jax.ShapeDtypeStruct((F, K // 8, B // 128, 8, 128),
                                      jnp.float32),
        compiler_params=pltpu.CompilerParams(
            use_tc_tiling_on_sc=False, needs_layout_passes=False),
        scratch_types=[
            pltpu.VMEM((CB,), jnp.int32),
            pltpu.VMEM((CB,), jnp.float32),
            pltpu.VMEM((CB, K), jnp.float32),
            pltpu.VMEM((K // 8, TC, 8, 128), jnp.float32),
            pltpu.SemaphoreType.DMA,
        ],
    )
    def sc_kernel(idx_hbm, val_hbm, table_hbm, out_hbm,
                  idx_v, val_v, rows_v, outp_v, sem):
        wid = lax.axis_index("s") * NC + lax.axis_index("c")
        b0 = wid * CB
        iota = lax.iota(jnp.int32, L)

        def fbody(f, carry):
            pltpu.sync_copy(idx_hbm.at[f, pl.ds(b0, CB)], idx_v)
            pltpu.sync_copy(val_hbm.at[f, pl.ds(b0, CB)], val_v)
            cps = [
                pltpu.async_copy(
                    table_hbm.at[idx_v.at[pl.ds(g * G, G)]],
                    rows_v.at[pl.ds(g * G, G)],
                    sem,
                )
                for g in range(n_sub)
            ]
            for cp in cps:
                cp.wait()
            def hbody(h, carry):
                tc_vec = jnp.full((L,), h, jnp.int32)
                for j in range(128 // L):
                    val16 = val_v[pl.ds(h * 128 + j * L, L)]
                    for t in range(L):
                        bl = j * L + t
                        x = (rows_v[h * 128 + bl, :]
                             * jnp.full((L,), val16[t]))
                        plsc.store_scatter(
                            outp_v,
                            [iota // 8, tc_vec, iota % 8,
                             jnp.full((L,), bl, jnp.int32)],
                            x)
                return carry

            lax.fori_loop(0, TC, hbody, 0)
            pltpu.sync_copy(
                outp_v,
                out_hbm.at[f, :, pl.ds(wid * TC, TC), :, :])
            return carry

        lax.fori_loop(0, F, fbody, 0)

    return sc_kernel


def kernel(nonzero_index, nonzero_value, table):
    B, F = nonzero_index.shape
    V, K = table.shape
    table_rows = table
    idxt = nonzero_index.T.astype(jnp.int32)
    valt = nonzero_value.T
    res = _build_gather(B, F, V, K)(idxt, valt, table_rows)
    out = res.transpose((2, 4, 0, 1, 3)).reshape(B, F, K)
    return out


# speedup vs baseline: 1.4956x; 1.0504x over previous
import functools

import jax
import jax.numpy as jnp
from jax import lax
from jax.experimental import pallas as pl
from jax.experimental.pallas import tpu as pltpu
from jax.experimental.pallas import tpu_sc as plsc

L = 16


@functools.lru_cache(maxsize=None)
def _build_gather(B, F, V, K):
    info = plsc.get_sparse_core_info()
    NC, NS = info.num_cores, info.num_subcores
    NW = NC * NS
    assert B % (NW * 128) == 0 and K == L
    CB = B // NW
    TC = CB // 128
    G = 128
    n_sub = CB // G

    mesh = plsc.VectorSubcoreMesh(core_axis_name="c", subcore_axis_name="s")

    @functools.partial(
        pl.kernel,
        mesh=mesh,
        out_type=jax.ShapeDtypeStruct((F, K // 8, B // 128, 8, 128),
                                      jnp.float32),
        compiler_params=pltpu.CompilerParams(
            use_tc_tiling_on_sc=False, needs_layout_passes=False),
        scratch_types=[
            pltpu.VMEM((CB,), jnp.int32),
            pltpu.VMEM((CB,), jnp.float32),
            pltpu.VMEM((CB, K), jnp.float32),
            pltpu.VMEM((K // 8, TC, 8, 128), jnp.float32),
            pltpu.SemaphoreType.DMA,
        ],
    )
    def sc_kernel(idx_hbm, val_hbm, table_hbm, out_hbm,
                  idx_v, val_v, rows_v, outp_v, sem):
        wid = lax.axis_index("s") * NC + lax.axis_index("c")
        b0 = wid * CB
        iota = lax.iota(jnp.int32, L)

        def fbody(f, carry):
            pltpu.sync_copy(idx_hbm.at[f, pl.ds(b0, CB)], idx_v)
            pltpu.sync_copy(val_hbm.at[f, pl.ds(b0, CB)], val_v)
            cps = [
                pltpu.async_copy(
                    table_hbm.at[idx_v.at[pl.ds(g * G, G)]],
                    rows_v.at[pl.ds(g * G, G)],
                    sem,
                )
                for g in range(n_sub)
            ]
            for cp in cps:
                cp.wait()
            for j in range(CB // L):
                r16 = j * L + iota
                val16 = val_v[pl.ds(j * L, L)]
                for k in range(K):
                    col = plsc.load_gather(
                        rows_v, [r16, jnp.full((L,), k, jnp.int32)])
                    outp_v[k // 8, j // 8, k % 8,
                           pl.ds((j % 8) * L, L)] = col * val16
            pltpu.sync_copy(
                outp_v,
                out_hbm.at[f, :, pl.ds(wid * TC, TC), :, :])
            return carry

        lax.fori_loop(0, F, fbody, 0)

    return sc_kernel


def kernel(nonzero_index, nonzero_value, table):
    B, F = nonzero_index.shape
    V, K = table.shape
    table_rows = table
    idxt = nonzero_index.T.astype(jnp.int32)
    valt = nonzero_value.T
    res = _build_gather(B, F, V, K)(idxt, valt, table_rows)
    out = res.transpose((2, 4, 0, 1, 3)).reshape(B, F, K)
    return out
